# baseline (speedup 1.0000x reference)
import functools

import jax
import jax.numpy as jnp
from jax import lax
from jax.experimental import pallas as pl
from jax.experimental.pallas import tpu as pltpu
from jax.experimental.pallas import tpu_sc as plsc

NC = 2
NS = 16
NW = NC * NS
L = 16
GB = 128
SB = 256

N_DRUG, N_GENE, D_IN = 10000, 50000, 128

_RELS = (("dd", 320000, N_DRUG), ("dg", 480000, N_GENE),
         ("gd", 480000, N_DRUG), ("gg", 640000, N_GENE))

ZR = 64


def _chunk_list(n_dst, dout):
    cr = 10016 if dout == 128 else 16672
    out, lo = [], 0
    while lo < n_dst:
        c = min(cr, n_dst - lo)
        out.append((lo, c))
        lo += c
    return out


def _make_sc_layer(dout):
    crmax = max(c for _, _, ndst in _RELS for _, c in _chunk_list(ndst, dout))
    trash = crmax
    mesh = plsc.VectorSubcoreMesh(core_axis_name="c", subcore_axis_name="s")
    out_type = [jax.ShapeDtypeStruct((NC, ndst, dout), jnp.float32)
                for _, _, ndst in _RELS]
    scratch = [
        pltpu.VMEM((SB,), jnp.int32),
        pltpu.VMEM((SB,), jnp.int32),
        [pltpu.VMEM((GB,), jnp.int32) for _ in range(SB // GB)],
        [pltpu.VMEM((GB,), jnp.int32) for _ in range(SB // GB)],
        pltpu.VMEM((SB, dout), jnp.float32),
        pltpu.VMEM((ZR, dout), jnp.float32),
        pltpu.VMEM_SHARED((crmax + L, dout), jnp.float32),
        pltpu.SemaphoreType.DMA,
    ]

    @functools.partial(
        pl.kernel, out_type=out_type, mesh=mesh, scratch_types=scratch,
        compiler_params=pltpu.CompilerParams(use_tc_tiling_on_sc=False))
    def body(ei_dd, m_dd, ei_dg, m_dg, ei_gd, m_gd, ei_gg, m_gg,
             p_dd, p_dg, p_gd, p_gg,
             sbat, dbat, sidx, lidx, rows, zeros, acc, sem):
        cid = lax.axis_index("c")
        sid = lax.axis_index("s")
        wid = sid * NC + cid
        iota = lax.iota(jnp.int32, L)

        for r in range(ZR):
            for k in range(dout // L):
                zeros[r, pl.ds(k * L, L)] = jnp.zeros((L,), jnp.float32)

        def proc_window(msg, lo, cr, lvs):
            gl = GB // L
            for k in range(SB // L):
                sv = sbat[pl.ds(k * L, L)]
                dv = dbat[pl.ds(k * L, L)]
                m = (k * L + iota >= lvs) & (dv >= lo) & (dv < lo + cr)
                sidx[k // gl][pl.ds((k % gl) * L, L)] = jnp.where(m, sv, 0)
                lidx[k // gl][pl.ds((k % gl) * L, L)] = jnp.where(
                    m, dv - lo, trash)
            descs = [
                pltpu.async_copy(msg.at[sidx[s]],
                                 rows.at[pl.ds(s * GB, GB)], sem)
                for s in range(SB // GB)]
            for d in descs:
                d.wait()
            for s in range(SB // GB):
                pltpu.sync_copy(rows.at[pl.ds(s * GB, GB)],
                                acc.at[lidx[s]], add=True)

        for (ei, msg, part), (_, E, ndst) in zip(
                ((ei_dd, m_dd, p_dd), (ei_dg, m_dg, p_dg),
                 (ei_gd, m_gd, p_gd), (ei_gg, m_gg, p_gg)), _RELS):
            strip = E // NW
            base = pl.multiple_of(wid * strip, 8)
            nwin = (strip + SB - 1) // SB
            for lo, cr in _chunk_list(ndst, dout):
                plsc.subcore_barrier()
                nzrows = cr + L
                nzfull, zrem = nzrows // ZR, nzrows % ZR
                for k in range((nzfull + NS - 1) // NS):
                    b = k * NS + sid
                    @pl.when(b < nzfull)
                    def _(b=b):
                        off = pl.multiple_of(b * ZR, ZR)
                        pltpu.sync_copy(zeros, acc.at[pl.ds(off, ZR)])
                if zrem:
                    @pl.when(sid == NS - 1)
                    def _(nzfull=nzfull, zrem=zrem):
                        pltpu.sync_copy(zeros.at[pl.ds(0, zrem)],
                                        acc.at[pl.ds(nzfull * ZR, zrem)])
                plsc.subcore_barrier()

                def bbody(j, carry, lo=lo, cr=cr, msg=msg, ei=ei, E=E,
                          base=base, strip=strip):
                    w = pl.multiple_of(
                        base + jnp.minimum(j * SB, strip - SB), 8)
                    lvs = base + j * SB - w
                    pltpu.sync_copy(ei.at[pl.ds(w, SB)], sbat)
                    pltpu.sync_copy(ei.at[pl.ds(E + w, SB)], dbat)
                    proc_window(msg, lo, cr, lvs)
                    return carry
                lax.fori_loop(0, nwin, bbody, 0)
                plsc.subcore_barrier()

                nofull, orem = cr // GB, cr % GB
                for k in range((nofull + NS - 1) // NS):
                    b = k * NS + sid
                    @pl.when(b < nofull)
                    def _(b=b, lo=lo, part=part):
                        off = pl.multiple_of(b * GB, GB)
                        pltpu.sync_copy(acc.at[pl.ds(off, GB)],
                                        part.at[cid, pl.ds(lo + off, GB)])
                if orem:
                    @pl.when(sid == NS - 1)
                    def _(lo=lo, part=part, nofull=nofull, orem=orem):
                        pltpu.sync_copy(
                            acc.at[pl.ds(nofull * GB, orem)],
                            part.at[cid, pl.ds(lo + nofull * GB, orem)])

    return body


def _mm(x, w, b, blk=2000):
    n, din = x.shape
    dout = w.shape[1]

    def kfn(x_ref, w_ref, b_ref, o_ref):
        o_ref[...] = jnp.dot(x_ref[...], w_ref[...],
                             preferred_element_type=jnp.float32,
                             precision=lax.Precision.HIGHEST) + b_ref[...]

    return pl.pallas_call(
        kfn,
        grid=(n // blk,),
        in_specs=[pl.BlockSpec((blk, din), lambda i: (i, 0)),
                  pl.BlockSpec((din, dout), lambda i: (0, 0)),
                  pl.BlockSpec((1, dout), lambda i: (0, 0))],
        out_specs=pl.BlockSpec((blk, dout), lambda i: (i, 0)),
        out_shape=jax.ShapeDtypeStruct((n, dout), jnp.float32),
    )(x, w, b.reshape(1, -1))


def _combine(x, wa, ba, pa, wb, bb, pb, relu, blk=2000):
    n, din = x.shape
    dout = wa.shape[1]

    def kfn(x_ref, wa_ref, ba_ref, pa_ref, wb_ref, bb_ref, pb_ref, o_ref):
        xx = x_ref[...]
        oa = (pa_ref[0] + pa_ref[1] + ba_ref[...] +
              jnp.dot(xx, wa_ref[...], preferred_element_type=jnp.float32,
                      precision=lax.Precision.HIGHEST))
        ob = (pb_ref[0] + pb_ref[1] + bb_ref[...] +
              jnp.dot(xx, wb_ref[...], preferred_element_type=jnp.float32,
                      precision=lax.Precision.HIGHEST))
        na = jnp.maximum(jnp.sqrt(jnp.sum(oa * oa, axis=1, keepdims=True)),
                         1e-12)
        nb_ = jnp.maximum(jnp.sqrt(jnp.sum(ob * ob, axis=1, keepdims=True)),
                          1e-12)
        out = oa / na + ob / nb_
        if relu:
            out = jnp.maximum(out, 0.0)
        o_ref[...] = out

    return pl.pallas_call(
        kfn,
        grid=(n // blk,),
        in_specs=[pl.BlockSpec((blk, din), lambda i: (i, 0)),
                  pl.BlockSpec((din, dout), lambda i: (0, 0)),
                  pl.BlockSpec((1, dout), lambda i: (0, 0)),
                  pl.BlockSpec((NC, blk, dout), lambda i: (0, i, 0)),
                  pl.BlockSpec((din, dout), lambda i: (0, 0)),
                  pl.BlockSpec((1, dout), lambda i: (0, 0)),
                  pl.BlockSpec((NC, blk, dout), lambda i: (0, i, 0))],
        out_specs=pl.BlockSpec((blk, dout), lambda i: (i, 0)),
        out_shape=jax.ShapeDtypeStruct((n, dout), jnp.float32),
    )(x, wa, ba.reshape(1, -1), pa, wb, bb.reshape(1, -1), pb)


def kernel(x_drug, x_gene, ei_drug_drug, ei_drug_gene, ei_gene_drug,
           ei_gene_gene,
           Wm0_drug_drug, bm0_drug_drug, Ws0_drug_drug, bs0_drug_drug,
           Wm0_drug_gene, bm0_drug_gene, Ws0_drug_gene, bs0_drug_gene,
           Wm0_gene_drug, bm0_gene_drug, Ws0_gene_drug, bs0_gene_drug,
           Wm0_gene_gene, bm0_gene_gene, Ws0_gene_gene, bs0_gene_gene,
           Wm1_drug_drug, bm1_drug_drug, Ws1_drug_drug, bs1_drug_drug,
           Wm1_drug_gene, bm1_drug_gene, Ws1_drug_gene, bs1_drug_gene,
           Wm1_gene_drug, bm1_gene_drug, Ws1_gene_drug, bs1_gene_drug,
           Wm1_gene_gene, bm1_gene_gene, Ws1_gene_gene, bs1_gene_gene):
    z_d, z_g = x_drug, x_gene
    layer_params = (
        (Wm0_drug_drug, bm0_drug_drug, Ws0_drug_drug, bs0_drug_drug,
         Wm0_drug_gene, bm0_drug_gene, Ws0_drug_gene, bs0_drug_gene,
         Wm0_gene_drug, bm0_gene_drug, Ws0_gene_drug, bs0_gene_drug,
         Wm0_gene_gene, bm0_gene_gene, Ws0_gene_gene, bs0_gene_gene),
        (Wm1_drug_drug, bm1_drug_drug, Ws1_drug_drug, bs1_drug_drug,
         Wm1_drug_gene, bm1_drug_gene, Ws1_drug_gene, bs1_drug_gene,
         Wm1_gene_drug, bm1_gene_drug, Ws1_gene_drug, bs1_gene_drug,
         Wm1_gene_gene, bm1_gene_gene, Ws1_gene_gene, bs1_gene_gene),
    )
    for li, dout in enumerate((128, 64)):
        (Wm_dd, bm_dd, Ws_dd, bs_dd, Wm_dg, bm_dg, Ws_dg, bs_dg,
         Wm_gd, bm_gd, Ws_gd, bs_gd, Wm_gg, bm_gg, Ws_gg, bs_gg) = (
            layer_params[li])
        m_dd = _mm(z_d, Wm_dd, bm_dd)
        m_dg = _mm(z_d, Wm_dg, bm_dg)
        m_gd = _mm(z_g, Wm_gd, bm_gd)
        m_gg = _mm(z_g, Wm_gg, bm_gg)
        sc = _make_sc_layer(dout)
        p_dd, p_dg, p_gd, p_gg = sc(
            ei_drug_drug.reshape(-1), m_dd, ei_drug_gene.reshape(-1), m_dg,
            ei_gene_drug.reshape(-1), m_gd, ei_gene_gene.reshape(-1), m_gg)
        relu = li == 0
        z_d_new = _combine(z_d, Ws_dd, bs_dd, p_dd, Ws_gd, bs_gd, p_gd, relu)
        z_g_new = _combine(z_g, Ws_dg, bs_dg, p_dg, Ws_gg, bs_gg, p_gg, relu)
        z_d, z_g = z_d_new, z_g_new
    return (z_d, z_g)


# speedup vs baseline: 34.4630x; 34.4630x over previous
import functools

import jax
import jax.numpy as jnp
from jax import lax
from jax.experimental import pallas as pl
from jax.experimental.pallas import tpu as pltpu
from jax.experimental.pallas import tpu_sc as plsc

NC = 2
NS = 16
NW = NC * NS
L = 16
GB = 128
SB = 256

N_DRUG, N_GENE, D_IN = 10000, 50000, 128

_RELS = (("dd", 320000, N_DRUG), ("dg", 480000, N_GENE),
         ("gd", 480000, N_DRUG), ("gg", 640000, N_GENE))

ZR = 64


def _chunk_list(n_dst, dout):
    cr = 10016 if dout == 128 else 16672
    out, lo = [], 0
    while lo < n_dst:
        c = min(cr, n_dst - lo)
        out.append((lo, c))
        lo += c
    return out


def _make_sc_layer(dout):
    crmax = max(c for _, _, ndst in _RELS for _, c in _chunk_list(ndst, dout))
    trash = crmax
    mesh = plsc.VectorSubcoreMesh(core_axis_name="c", subcore_axis_name="s")
    out_type = [jax.ShapeDtypeStruct((NC, ndst, dout), jnp.float32)
                for _, _, ndst in _RELS]
    scratch = [
        pltpu.VMEM((SB,), jnp.int32),
        pltpu.VMEM((SB,), jnp.int32),
        [pltpu.VMEM((GB,), jnp.int32) for _ in range(SB // GB)],
        [pltpu.VMEM((GB,), jnp.int32) for _ in range(SB // GB)],
        pltpu.VMEM((SB, dout), jnp.float32),
        pltpu.VMEM((ZR, dout), jnp.float32),
        pltpu.VMEM_SHARED((crmax + L, dout), jnp.float32),
        pltpu.SemaphoreType.DMA,
    ]

    @functools.partial(
        pl.kernel, out_type=out_type, mesh=mesh, scratch_types=scratch,
        compiler_params=pltpu.CompilerParams(use_tc_tiling_on_sc=False))
    def body(ei_dd, m_dd, ei_dg, m_dg, ei_gd, m_gd, ei_gg, m_gg,
             p_dd, p_dg, p_gd, p_gg,
             sbat, dbat, sidx, lidx, rows, zeros, acc, sem):
        cid = lax.axis_index("c")
        sid = lax.axis_index("s")
        wid = sid * NC + cid
        iota = lax.iota(jnp.int32, L)

        for r in range(ZR):
            for k in range(dout // L):
                zeros[r, pl.ds(k * L, L)] = jnp.zeros((L,), jnp.float32)

        def proc_window(msg, lo, cr, lvs):
            gl = GB // L
            for k in range(SB // L):
                sv = sbat[pl.ds(k * L, L)]
                dv = dbat[pl.ds(k * L, L)]
                m = (k * L + iota >= lvs) & (dv >= lo) & (dv < lo + cr)
                sidx[k // gl][pl.ds((k % gl) * L, L)] = sv
                lidx[k // gl][pl.ds((k % gl) * L, L)] = jnp.where(
                    m, dv - lo, trash)
            descs = [
                pltpu.async_copy(msg.at[sidx[s]],
                                 rows.at[pl.ds(s * GB, GB)], sem)
                for s in range(SB // GB)]
            for d in descs:
                d.wait()
            for s in range(SB // GB):
                pltpu.sync_copy(rows.at[pl.ds(s * GB, GB)],
                                acc.at[lidx[s]], add=True)

        for (ei, msg, part), (_, E, ndst) in zip(
                ((ei_dd, m_dd, p_dd), (ei_dg, m_dg, p_dg),
                 (ei_gd, m_gd, p_gd), (ei_gg, m_gg, p_gg)), _RELS):
            strip = E // NW
            base = pl.multiple_of(wid * strip, 8)
            nwin = (strip + SB - 1) // SB
            for lo, cr in _chunk_list(ndst, dout):
                plsc.subcore_barrier()
                nzrows = cr + L
                nzfull, zrem = nzrows // ZR, nzrows % ZR
                for k in range((nzfull + NS - 1) // NS):
                    b = k * NS + sid
                    @pl.when(b < nzfull)
                    def _(b=b):
                        off = pl.multiple_of(b * ZR, ZR)
                        pltpu.sync_copy(zeros, acc.at[pl.ds(off, ZR)])
                if zrem:
                    @pl.when(sid == NS - 1)
                    def _(nzfull=nzfull, zrem=zrem):
                        pltpu.sync_copy(zeros.at[pl.ds(0, zrem)],
                                        acc.at[pl.ds(nzfull * ZR, zrem)])
                plsc.subcore_barrier()

                def bbody(j, carry, lo=lo, cr=cr, msg=msg, ei=ei, E=E,
                          base=base, strip=strip):
                    w = pl.multiple_of(
                        base + jnp.minimum(j * SB, strip - SB), 8)
                    lvs = base + j * SB - w
                    pltpu.sync_copy(ei.at[pl.ds(w, SB)], sbat)
                    pltpu.sync_copy(ei.at[pl.ds(E + w, SB)], dbat)
                    proc_window(msg, lo, cr, lvs)
                    return carry
                lax.fori_loop(0, nwin, bbody, 0)
                plsc.subcore_barrier()

                nofull, orem = cr // GB, cr % GB
                for k in range((nofull + NS - 1) // NS):
                    b = k * NS + sid
                    @pl.when(b < nofull)
                    def _(b=b, lo=lo, part=part):
                        off = pl.multiple_of(b * GB, GB)
                        pltpu.sync_copy(acc.at[pl.ds(off, GB)],
                                        part.at[cid, pl.ds(lo + off, GB)])
                if orem:
                    @pl.when(sid == NS - 1)
                    def _(lo=lo, part=part, nofull=nofull, orem=orem):
                        pltpu.sync_copy(
                            acc.at[pl.ds(nofull * GB, orem)],
                            part.at[cid, pl.ds(lo + nofull * GB, orem)])

    return body


def _mm(x, w, b, blk=2000):
    n, din = x.shape
    dout = w.shape[1]

    def kfn(x_ref, w_ref, b_ref, o_ref):
        o_ref[...] = jnp.dot(x_ref[...], w_ref[...],
                             preferred_element_type=jnp.float32,
                             precision=lax.Precision.HIGHEST) + b_ref[...]

    return pl.pallas_call(
        kfn,
        grid=(n // blk,),
        in_specs=[pl.BlockSpec((blk, din), lambda i: (i, 0)),
                  pl.BlockSpec((din, dout), lambda i: (0, 0)),
                  pl.BlockSpec((1, dout), lambda i: (0, 0))],
        out_specs=pl.BlockSpec((blk, dout), lambda i: (i, 0)),
        out_shape=jax.ShapeDtypeStruct((n, dout), jnp.float32),
    )(x, w, b.reshape(1, -1))


def _combine(x, wa, ba, pa, wb, bb, pb, relu, blk=2000):
    n, din = x.shape
    dout = wa.shape[1]

    def kfn(x_ref, wa_ref, ba_ref, pa_ref, wb_ref, bb_ref, pb_ref, o_ref):
        xx = x_ref[...]
        oa = (pa_ref[0] + pa_ref[1] + ba_ref[...] +
              jnp.dot(xx, wa_ref[...], preferred_element_type=jnp.float32,
                      precision=lax.Precision.HIGHEST))
        ob = (pb_ref[0] + pb_ref[1] + bb_ref[...] +
              jnp.dot(xx, wb_ref[...], preferred_element_type=jnp.float32,
                      precision=lax.Precision.HIGHEST))
        na = jnp.maximum(jnp.sqrt(jnp.sum(oa * oa, axis=1, keepdims=True)),
                         1e-12)
        nb_ = jnp.maximum(jnp.sqrt(jnp.sum(ob * ob, axis=1, keepdims=True)),
                          1e-12)
        out = oa / na + ob / nb_
        if relu:
            out = jnp.maximum(out, 0.0)
        o_ref[...] = out

    return pl.pallas_call(
        kfn,
        grid=(n // blk,),
        in_specs=[pl.BlockSpec((blk, din), lambda i: (i, 0)),
                  pl.BlockSpec((din, dout), lambda i: (0, 0)),
                  pl.BlockSpec((1, dout), lambda i: (0, 0)),
                  pl.BlockSpec((NC, blk, dout), lambda i: (0, i, 0)),
                  pl.BlockSpec((din, dout), lambda i: (0, 0)),
                  pl.BlockSpec((1, dout), lambda i: (0, 0)),
                  pl.BlockSpec((NC, blk, dout), lambda i: (0, i, 0))],
        out_specs=pl.BlockSpec((blk, dout), lambda i: (i, 0)),
        out_shape=jax.ShapeDtypeStruct((n, dout), jnp.float32),
    )(x, wa, ba.reshape(1, -1), pa, wb, bb.reshape(1, -1), pb)


def kernel(x_drug, x_gene, ei_drug_drug, ei_drug_gene, ei_gene_drug,
           ei_gene_gene,
           Wm0_drug_drug, bm0_drug_drug, Ws0_drug_drug, bs0_drug_drug,
           Wm0_drug_gene, bm0_drug_gene, Ws0_drug_gene, bs0_drug_gene,
           Wm0_gene_drug, bm0_gene_drug, Ws0_gene_drug, bs0_gene_drug,
           Wm0_gene_gene, bm0_gene_gene, Ws0_gene_gene, bs0_gene_gene,
           Wm1_drug_drug, bm1_drug_drug, Ws1_drug_drug, bs1_drug_drug,
           Wm1_drug_gene, bm1_drug_gene, Ws1_drug_gene, bs1_drug_gene,
           Wm1_gene_drug, bm1_gene_drug, Ws1_gene_drug, bs1_gene_drug,
           Wm1_gene_gene, bm1_gene_gene, Ws1_gene_gene, bs1_gene_gene):
    z_d, z_g = x_drug, x_gene
    layer_params = (
        (Wm0_drug_drug, bm0_drug_drug, Ws0_drug_drug, bs0_drug_drug,
         Wm0_drug_gene, bm0_drug_gene, Ws0_drug_gene, bs0_drug_gene,
         Wm0_gene_drug, bm0_gene_drug, Ws0_gene_drug, bs0_gene_drug,
         Wm0_gene_gene, bm0_gene_gene, Ws0_gene_gene, bs0_gene_gene),
        (Wm1_drug_drug, bm1_drug_drug, Ws1_drug_drug, bs1_drug_drug,
         Wm1_drug_gene, bm1_drug_gene, Ws1_drug_gene, bs1_drug_gene,
         Wm1_gene_drug, bm1_gene_drug, Ws1_gene_drug, bs1_gene_drug,
         Wm1_gene_gene, bm1_gene_gene, Ws1_gene_gene, bs1_gene_gene),
    )
    for li, dout in enumerate((128, 64)):
        (Wm_dd, bm_dd, Ws_dd, bs_dd, Wm_dg, bm_dg, Ws_dg, bs_dg,
         Wm_gd, bm_gd, Ws_gd, bs_gd, Wm_gg, bm_gg, Ws_gg, bs_gg) = (
            layer_params[li])
        m_dd = _mm(z_d, Wm_dd, bm_dd)
        m_dg = _mm(z_d, Wm_dg, bm_dg)
        m_gd = _mm(z_g, Wm_gd, bm_gd)
        m_gg = _mm(z_g, Wm_gg, bm_gg)
        sc = _make_sc_layer(dout)
        p_dd, p_dg, p_gd, p_gg = sc(
            ei_drug_drug.reshape(-1), m_dd, ei_drug_gene.reshape(-1), m_dg,
            ei_gene_drug.reshape(-1), m_gd, ei_gene_gene.reshape(-1), m_gg)
        relu = li == 0
        z_d_new = _combine(z_d, Ws_dd, bs_dd, p_dd, Ws_gd, bs_gd, p_gd, relu)
        z_g_new = _combine(z_g, Ws_dg, bs_dg, p_dg, Ws_gg, bs_gg, p_gg, relu)
        z_d, z_g = z_d_new, z_g_new
    return (z_d, z_g)


# speedup vs baseline: 34.5281x; 1.0019x over previous
import functools

import jax
import jax.numpy as jnp
from jax import lax
from jax.experimental import pallas as pl
from jax.experimental.pallas import tpu as pltpu
from jax.experimental.pallas import tpu_sc as plsc

NC = 2
NS = 16
NW = NC * NS
L = 16
GB = 128
SB = 128

N_DRUG, N_GENE, D_IN = 10000, 50000, 128

_RELS = (("dd", 320000, N_DRUG), ("dg", 480000, N_GENE),
         ("gd", 480000, N_DRUG), ("gg", 640000, N_GENE))

ZR = 16


def _chunk_list(n_dst, dout):
    cr = 12512 if dout == 128 else 25008
    out, lo = [], 0
    while lo < n_dst:
        c = min(cr, n_dst - lo)
        out.append((lo, c))
        lo += c
    return out


def _make_sc_layer(dout):
    crmax = max(c for _, _, ndst in _RELS for _, c in _chunk_list(ndst, dout))
    trash = crmax
    mesh = plsc.VectorSubcoreMesh(core_axis_name="c", subcore_axis_name="s")
    out_type = [jax.ShapeDtypeStruct((NC, ndst, dout), jnp.float32)
                for _, _, ndst in _RELS]
    scratch = [
        pltpu.VMEM((SB,), jnp.int32),
        pltpu.VMEM((SB,), jnp.int32),
        [pltpu.VMEM((GB,), jnp.int32) for _ in range(SB // GB)],
        [pltpu.VMEM((GB,), jnp.int32) for _ in range(SB // GB)],
        pltpu.VMEM((SB, dout), jnp.float32),
        pltpu.VMEM((ZR, dout), jnp.float32),
        pltpu.VMEM_SHARED((crmax + L, dout), jnp.float32),
        pltpu.SemaphoreType.DMA,
    ]

    @functools.partial(
        pl.kernel, out_type=out_type, mesh=mesh, scratch_types=scratch,
        compiler_params=pltpu.CompilerParams(use_tc_tiling_on_sc=False))
    def body(ei_dd, m_dd, ei_dg, m_dg, ei_gd, m_gd, ei_gg, m_gg,
             p_dd, p_dg, p_gd, p_gg,
             sbat, dbat, sidx, lidx, rows, zeros, acc, sem):
        cid = lax.axis_index("c")
        sid = lax.axis_index("s")
        wid = sid * NC + cid
        iota = lax.iota(jnp.int32, L)

        for r in range(ZR):
            for k in range(dout // L):
                zeros[r, pl.ds(k * L, L)] = jnp.zeros((L,), jnp.float32)

        def proc_window(msg, lo, cr, lvs):
            gl = GB // L
            for k in range(SB // L):
                sv = sbat[pl.ds(k * L, L)]
                dv = dbat[pl.ds(k * L, L)]
                m = (k * L + iota >= lvs) & (dv >= lo) & (dv < lo + cr)
                sidx[k // gl][pl.ds((k % gl) * L, L)] = sv
                lidx[k // gl][pl.ds((k % gl) * L, L)] = jnp.where(
                    m, dv - lo, trash)
            descs = [
                pltpu.async_copy(msg.at[sidx[s]],
                                 rows.at[pl.ds(s * GB, GB)], sem)
                for s in range(SB // GB)]
            for d in descs:
                d.wait()
            for s in range(SB // GB):
                pltpu.sync_copy(rows.at[pl.ds(s * GB, GB)],
                                acc.at[lidx[s]], add=True)

        for (ei, msg, part), (_, E, ndst) in zip(
                ((ei_dd, m_dd, p_dd), (ei_dg, m_dg, p_dg),
                 (ei_gd, m_gd, p_gd), (ei_gg, m_gg, p_gg)), _RELS):
            strip = E // NW
            base = pl.multiple_of(wid * strip, 8)
            nwin = (strip + SB - 1) // SB
            for lo, cr in _chunk_list(ndst, dout):
                plsc.subcore_barrier()
                nzrows = cr + L
                nzfull, zrem = nzrows // ZR, nzrows % ZR
                for k in range((nzfull + NS - 1) // NS):
                    b = k * NS + sid
                    @pl.when(b < nzfull)
                    def _(b=b):
                        off = pl.multiple_of(b * ZR, ZR)
                        pltpu.sync_copy(zeros, acc.at[pl.ds(off, ZR)])
                if zrem:
                    @pl.when(sid == NS - 1)
                    def _(nzfull=nzfull, zrem=zrem):
                        pltpu.sync_copy(zeros.at[pl.ds(0, zrem)],
                                        acc.at[pl.ds(nzfull * ZR, zrem)])
                plsc.subcore_barrier()

                def bbody(j, carry, lo=lo, cr=cr, msg=msg, ei=ei, E=E,
                          base=base, strip=strip):
                    w = pl.multiple_of(
                        base + jnp.minimum(j * SB, strip - SB), 8)
                    lvs = base + j * SB - w
                    pltpu.sync_copy(ei.at[pl.ds(w, SB)], sbat)
                    pltpu.sync_copy(ei.at[pl.ds(E + w, SB)], dbat)
                    proc_window(msg, lo, cr, lvs)
                    return carry
                lax.fori_loop(0, nwin, bbody, 0)
                plsc.subcore_barrier()

                nofull, orem = cr // GB, cr % GB
                for k in range((nofull + NS - 1) // NS):
                    b = k * NS + sid
                    @pl.when(b < nofull)
                    def _(b=b, lo=lo, part=part):
                        off = pl.multiple_of(b * GB, GB)
                        pltpu.sync_copy(acc.at[pl.ds(off, GB)],
                                        part.at[cid, pl.ds(lo + off, GB)])
                if orem:
                    @pl.when(sid == NS - 1)
                    def _(lo=lo, part=part, nofull=nofull, orem=orem):
                        pltpu.sync_copy(
                            acc.at[pl.ds(nofull * GB, orem)],
                            part.at[cid, pl.ds(lo + nofull * GB, orem)])

    return body


def _mm(x, w, b, blk=2000):
    n, din = x.shape
    dout = w.shape[1]

    def kfn(x_ref, w_ref, b_ref, o_ref):
        o_ref[...] = jnp.dot(x_ref[...], w_ref[...],
                             preferred_element_type=jnp.float32,
                             precision=lax.Precision.HIGHEST) + b_ref[...]

    return pl.pallas_call(
        kfn,
        grid=(n // blk,),
        in_specs=[pl.BlockSpec((blk, din), lambda i: (i, 0)),
                  pl.BlockSpec((din, dout), lambda i: (0, 0)),
                  pl.BlockSpec((1, dout), lambda i: (0, 0))],
        out_specs=pl.BlockSpec((blk, dout), lambda i: (i, 0)),
        out_shape=jax.ShapeDtypeStruct((n, dout), jnp.float32),
    )(x, w, b.reshape(1, -1))


def _combine(x, wa, ba, pa, wb, bb, pb, relu, blk=2000):
    n, din = x.shape
    dout = wa.shape[1]

    def kfn(x_ref, wa_ref, ba_ref, pa_ref, wb_ref, bb_ref, pb_ref, o_ref):
        xx = x_ref[...]
        oa = (pa_ref[0] + pa_ref[1] + ba_ref[...] +
              jnp.dot(xx, wa_ref[...], preferred_element_type=jnp.float32,
                      precision=lax.Precision.HIGHEST))
        ob = (pb_ref[0] + pb_ref[1] + bb_ref[...] +
              jnp.dot(xx, wb_ref[...], preferred_element_type=jnp.float32,
                      precision=lax.Precision.HIGHEST))
        na = jnp.maximum(jnp.sqrt(jnp.sum(oa * oa, axis=1, keepdims=True)),
                         1e-12)
        nb_ = jnp.maximum(jnp.sqrt(jnp.sum(ob * ob, axis=1, keepdims=True)),
                          1e-12)
        out = oa / na + ob / nb_
        if relu:
            out = jnp.maximum(out, 0.0)
        o_ref[...] = out

    return pl.pallas_call(
        kfn,
        grid=(n // blk,),
        in_specs=[pl.BlockSpec((blk, din), lambda i: (i, 0)),
                  pl.BlockSpec((din, dout), lambda i: (0, 0)),
                  pl.BlockSpec((1, dout), lambda i: (0, 0)),
                  pl.BlockSpec((NC, blk, dout), lambda i: (0, i, 0)),
                  pl.BlockSpec((din, dout), lambda i: (0, 0)),
                  pl.BlockSpec((1, dout), lambda i: (0, 0)),
                  pl.BlockSpec((NC, blk, dout), lambda i: (0, i, 0))],
        out_specs=pl.BlockSpec((blk, dout), lambda i: (i, 0)),
        out_shape=jax.ShapeDtypeStruct((n, dout), jnp.float32),
    )(x, wa, ba.reshape(1, -1), pa, wb, bb.reshape(1, -1), pb)


def kernel(x_drug, x_gene, ei_drug_drug, ei_drug_gene, ei_gene_drug,
           ei_gene_gene,
           Wm0_drug_drug, bm0_drug_drug, Ws0_drug_drug, bs0_drug_drug,
           Wm0_drug_gene, bm0_drug_gene, Ws0_drug_gene, bs0_drug_gene,
           Wm0_gene_drug, bm0_gene_drug, Ws0_gene_drug, bs0_gene_drug,
           Wm0_gene_gene, bm0_gene_gene, Ws0_gene_gene, bs0_gene_gene,
           Wm1_drug_drug, bm1_drug_drug, Ws1_drug_drug, bs1_drug_drug,
           Wm1_drug_gene, bm1_drug_gene, Ws1_drug_gene, bs1_drug_gene,
           Wm1_gene_drug, bm1_gene_drug, Ws1_gene_drug, bs1_gene_drug,
           Wm1_gene_gene, bm1_gene_gene, Ws1_gene_gene, bs1_gene_gene):
    z_d, z_g = x_drug, x_gene
    layer_params = (
        (Wm0_drug_drug, bm0_drug_drug, Ws0_drug_drug, bs0_drug_drug,
         Wm0_drug_gene, bm0_drug_gene, Ws0_drug_gene, bs0_drug_gene,
         Wm0_gene_drug, bm0_gene_drug, Ws0_gene_drug, bs0_gene_drug,
         Wm0_gene_gene, bm0_gene_gene, Ws0_gene_gene, bs0_gene_gene),
        (Wm1_drug_drug, bm1_drug_drug, Ws1_drug_drug, bs1_drug_drug,
         Wm1_drug_gene, bm1_drug_gene, Ws1_drug_gene, bs1_drug_gene,
         Wm1_gene_drug, bm1_gene_drug, Ws1_gene_drug, bs1_gene_drug,
         Wm1_gene_gene, bm1_gene_gene, Ws1_gene_gene, bs1_gene_gene),
    )
    for li, dout in enumerate((128, 64)):
        (Wm_dd, bm_dd, Ws_dd, bs_dd, Wm_dg, bm_dg, Ws_dg, bs_dg,
         Wm_gd, bm_gd, Ws_gd, bs_gd, Wm_gg, bm_gg, Ws_gg, bs_gg) = (
            layer_params[li])
        m_dd = _mm(z_d, Wm_dd, bm_dd)
        m_dg = _mm(z_d, Wm_dg, bm_dg)
        m_gd = _mm(z_g, Wm_gd, bm_gd)
        m_gg = _mm(z_g, Wm_gg, bm_gg)
        sc = _make_sc_layer(dout)
        p_dd, p_dg, p_gd, p_gg = sc(
            ei_drug_drug.reshape(-1), m_dd, ei_drug_gene.reshape(-1), m_dg,
            ei_gene_drug.reshape(-1), m_gd, ei_gene_gene.reshape(-1), m_gg)
        relu = li == 0
        z_d_new = _combine(z_d, Ws_dd, bs_dd, p_dd, Ws_gd, bs_gd, p_gd, relu)
        z_g_new = _combine(z_g, Ws_dg, bs_dg, p_dg, Ws_gg, bs_gg, p_gg, relu)
        z_d, z_g = z_d_new, z_g_new
    return (z_d, z_g)
